# baseline (speedup 1.0000x reference)
import functools

import jax
import jax.numpy as jnp
from jax import lax
from jax.experimental import pallas as pl
from jax.experimental.pallas import tpu as pltpu
from jax.experimental.pallas import tpu_sc as plsc

P = 4096
SEQ = 4096
STATE = 2048
L = 16
CH = 16
U = 8
NVEC = SEQ // L
NCHUNK = 2
NR = STATE // NCHUNK
TC_BLK = 256


def _make_sc_energies(r0):
    def body(hist_hbm, cur_hbm, dist_hbm, e_hbm,
             hist_v, cur_all_v, idx16_v, rows_v, ea_v, eb_v,
             sem_in, sem_a, sem_b):
        info = plsc.get_sparse_core_info()
        nc, ns = info.num_cores, info.num_subcores
        nw = nc * ns
        rows_per_w = NR // nw
        wid = lax.axis_index("s") * nc + lax.axis_index("c")

        pltpu.sync_copy(hist_hbm, hist_v)
        pltpu.sync_copy(cur_hbm, cur_all_v)

        iota = lax.iota(jnp.int32, L)

        def gather_row(t, e_ref):
            c = t >> 4
            k = t - (c << 4)
            i = r0 + wid + t * nw
            kvec = jnp.full((L,), k, jnp.int32)

            @pl.when(k == 0)
            def _():
                rowidx = plsc.load_gather(
                    cur_all_v, [(r0 + wid) + (c * CH + iota) * nw])
                idx16_v[pl.ds(0, L)] = rowidx
                pltpu.async_copy(
                    dist_hbm.at[idx16_v], rows_v, sem_in).wait()

            nv2 = (((i + 1) >> 7) << 3) + 16

            @plsc.parallel_loop(0, nv2, unroll=U)
            def _(v):
                idx = hist_v[pl.ds(v * L, L)]
                g = plsc.load_gather(rows_v, [kvec, idx])
                e_ref[pl.ds(v * L, L)] = 1.0 / g

            return i - r0

        def pair_body(q, carry):
            ia = gather_row(2 * q, ea_v)
            pltpu.async_copy(ea_v, e_hbm.at[ia], sem_a)

            @pl.when(q > 0)
            def _():
                pltpu.make_async_copy(eb_v, e_hbm.at[ia], sem_b).wait()

            ib = gather_row(2 * q + 1, eb_v)
            pltpu.async_copy(eb_v, e_hbm.at[ib], sem_b)
            pltpu.make_async_copy(ea_v, e_hbm.at[ia], sem_a).wait()
            return carry

        lax.fori_loop(0, NR // nw // 2, pair_body, 0)
        pltpu.make_async_copy(eb_v, e_hbm.at[0], sem_b).wait()

    return functools.partial(
        pl.kernel,
        out_type=jax.ShapeDtypeStruct((NR, SEQ), jnp.float32),
        mesh=plsc.VectorSubcoreMesh(
            core_axis_name="c", subcore_axis_name="s"),
        compiler_params=pltpu.CompilerParams(
            use_tc_tiling_on_sc=True, needs_layout_passes=False),
        scratch_types=[
            pltpu.VMEM((SEQ,), jnp.int32),
            pltpu.VMEM((STATE,), jnp.int32),
            pltpu.VMEM((L,), jnp.int32),
            pltpu.VMEM((CH, SEQ), jnp.float32),
            pltpu.VMEM((SEQ,), jnp.float32),
            pltpu.VMEM((SEQ,), jnp.float32),
            pltpu.SemaphoreType.DMA,
            pltpu.SemaphoreType.DMA,
            pltpu.SemaphoreType.DMA,
        ],
    )(body)


_sc_chunk = [_make_sc_energies(c * NR) for c in range(NCHUNK)]


def _make_tc_body(r0):
    def body(e_ref, *rest):
        o_ref = rest[-1]
        b = pl.program_id(0)
        rows = (jax.lax.broadcasted_iota(jnp.int32, (TC_BLK, SEQ), 0)
                + b * TC_BLK + r0)
        cols = jax.lax.broadcasted_iota(jnp.int32, (TC_BLK, SEQ), 1)
        e = jnp.where(cols <= rows, e_ref[...], 0.0)
        m = jnp.max(e, axis=1, keepdims=True)
        p = jnp.exp(e - m)
        s = jnp.sum(p, axis=1, keepdims=True)
        o_ref[...] = p * (1.0 / s)
    return body


def _tc_softmax_chunk(c, e, prev):
    r0 = c * NR
    grid = (NR // TC_BLK,)
    in_specs = [pl.BlockSpec((TC_BLK, SEQ), lambda b: (b, 0))]
    operands = [e]
    aliases = {}
    if prev is not None:
        in_specs.append(pl.BlockSpec(memory_space=pl.ANY))
        operands.append(prev)
        aliases = {1: 0}
    off = r0 // TC_BLK
    return pl.pallas_call(
        _make_tc_body(r0),
        grid=grid,
        in_specs=in_specs,
        out_specs=pl.BlockSpec((TC_BLK, SEQ), lambda b: (b + off, 0)),
        out_shape=jax.ShapeDtypeStruct((STATE, SEQ), jnp.float32),
        input_output_aliases=aliases,
    )(*operands)


def kernel(history, current, poi_distance_matrix):
    hist = history.astype(jnp.int32)
    cur = current.astype(jnp.int32)
    out = None
    for c in range(NCHUNK):
        e = _sc_chunk[c](hist, cur, poi_distance_matrix)
        out = _tc_softmax_chunk(c, e, out)
    return out
